# baseline (speedup 1.0000x reference)
import functools

import jax
import jax.numpy as jnp
from jax import lax
from jax.experimental import pallas as pl
from jax.experimental.pallas import tpu as pltpu
from jax.experimental.pallas import tpu_sc as plsc

VOCAB_SIZE = 1000000
H = 32
B = 16384

_info = plsc.get_sparse_core_info()
_NC, _NS, _L = _info.num_cores, _info.num_subcores, _info.num_lanes
_NW = _NC * _NS
_BPW = B // _NW

_mesh = plsc.VectorSubcoreMesh(core_axis_name="c", subcore_axis_name="s")


@functools.partial(
    pl.kernel,
    mesh=_mesh,
    out_type=jax.ShapeDtypeStruct((B, H), jnp.float32),
    scratch_types=[
        pltpu.VMEM((_BPW,), jnp.int32),
        pltpu.VMEM((_BPW, H), jnp.float32),
        pltpu.SemaphoreType.DMA,
    ],
    compiler_params=pltpu.CompilerParams(use_tc_tiling_on_sc=False),
)
def _sc_gather(table_hbm, idx_hbm, out_hbm, idx_v, rows_v, sem):
    wid = lax.axis_index("s") * _NC + lax.axis_index("c")
    base = wid * _BPW
    pltpu.sync_copy(idx_hbm.at[pl.ds(base, _BPW)], idx_v)
    pltpu.async_copy(table_hbm.at[idx_v], rows_v, sem).wait()
    pltpu.sync_copy(rows_v, out_hbm.at[pl.ds(base, _BPW)])


def _mm_body(e_ref, w_ref, b_ref, o_ref):
    o_ref[...] = (
        jnp.dot(e_ref[...], w_ref[...], preferred_element_type=jnp.float32)
        + b_ref[...]
    )


def kernel(x, table, W, b):
    idx = x.reshape(B).astype(jnp.int32)
    e = _sc_gather(table, idx)
    blk = 2048
    out = pl.pallas_call(
        _mm_body,
        out_shape=jax.ShapeDtypeStruct((B, H), jnp.float32),
        grid=(B // blk,),
        in_specs=[
            pl.BlockSpec((blk, H), lambda i: (i, 0)),
            pl.BlockSpec((H, H), lambda i: (0, 0)),
            pl.BlockSpec((1, H), lambda i: (0, 0)),
        ],
        out_specs=pl.BlockSpec((blk, H), lambda i: (i, 0)),
    )(e, W, b.reshape(1, H))
    return out


# speedup vs baseline: 1.6779x; 1.6779x over previous
import functools

import jax
import jax.numpy as jnp
from jax import lax
from jax.experimental import pallas as pl
from jax.experimental.pallas import tpu as pltpu
from jax.experimental.pallas import tpu_sc as plsc

VOCAB_SIZE = 1000000
H = 32
B = 16384

_info = plsc.get_sparse_core_info()
_NC, _NS, _L = _info.num_cores, _info.num_subcores, _info.num_lanes
_NW = _NC * _NS
_BPW = B // _NW
_K = 32
_LAG = 4

_mesh = plsc.VectorSubcoreMesh(core_axis_name="c", subcore_axis_name="s")


@functools.partial(
    pl.kernel,
    mesh=_mesh,
    out_type=jax.ShapeDtypeStruct((B, H), jnp.float32),
    scratch_types=[
        pltpu.SMEM((_BPW,), jnp.int32),
        pltpu.VMEM_SHARED((B,), jnp.int32),
        pltpu.VMEM((_BPW, H), jnp.float32),
        pltpu.SemaphoreType.DMA,
    ],
)
def _sc_gather(table_hbm, idx_hbm, out_hbm, idx_s, idx_sp, rows_v, sem):
    wid = lax.axis_index("s") * _NC + lax.axis_index("c")
    base = wid * _BPW
    pltpu.sync_copy(idx_hbm.at[pl.ds(base, _BPW)], idx_sp.at[pl.ds(base, _BPW)])
    pltpu.sync_copy(idx_sp.at[pl.ds(base, _BPW)], idx_s)

    n_groups = _BPW // _K

    def issue(g):
        for j in range(_K):
            i = g * _K + j
            pltpu.async_copy(
                table_hbm.at[pl.ds(idx_s[i], 1)], rows_v.at[pl.ds(i, 1)], sem
            )

    def drain(g):
        pltpu.make_async_copy(
            table_hbm.at[pl.ds(0, _K)], rows_v.at[pl.ds(g * _K, _K)], sem
        ).wait()

    for g in range(_LAG):
        issue(g)

    def body(g, _):
        issue(g)
        drain(g - _LAG)
        return ()

    lax.fori_loop(_LAG, n_groups, body, (), unroll=False)
    for g in range(n_groups - _LAG, n_groups):
        drain(g)
    pltpu.sync_copy(rows_v, out_hbm.at[pl.ds(base, _BPW)])


def _mm_body(w_ref, e_ref, b_ref, o_ref):
    o_ref[...] = (
        lax.dot_general(
            w_ref[...], e_ref[...],
            (((0,), (1,)), ((), ())),
            preferred_element_type=jnp.float32,
        )
        + b_ref[...]
    )


def kernel(x, table, W, b):
    idx = x.reshape(B).astype(jnp.int32)
    e = _sc_gather(table, idx)
    blk = 2048
    out_t = pl.pallas_call(
        _mm_body,
        out_shape=jax.ShapeDtypeStruct((H, B), jnp.float32),
        grid=(B // blk,),
        in_specs=[
            pl.BlockSpec((H, H), lambda i: (0, 0)),
            pl.BlockSpec((blk, H), lambda i: (i, 0)),
            pl.BlockSpec((H, 1), lambda i: (0, 0)),
        ],
        out_specs=pl.BlockSpec((H, blk), lambda i: (0, i)),
    )(W, e, b.reshape(H, 1))
    return out_t.T


# speedup vs baseline: 1.6982x; 1.0121x over previous
import functools

import jax
import jax.numpy as jnp
from jax import lax
from jax.experimental import pallas as pl
from jax.experimental.pallas import tpu as pltpu
from jax.experimental.pallas import tpu_sc as plsc

VOCAB_SIZE = 1000000
H = 32
B = 16384

_info = plsc.get_sparse_core_info()
_NC, _NS, _L = _info.num_cores, _info.num_subcores, _info.num_lanes
_NW = _NC * _NS
_BPW = B // _NW
_K = 32
_LAG = 4

_mesh = plsc.VectorSubcoreMesh(core_axis_name="c", subcore_axis_name="s")


@functools.partial(
    pl.kernel,
    mesh=_mesh,
    out_type=jax.ShapeDtypeStruct((B, H), jnp.float32),
    scratch_types=[
        pltpu.SMEM((_BPW,), jnp.int32),
        pltpu.VMEM_SHARED((B,), jnp.int32),
        pltpu.VMEM((_BPW, H), jnp.float32),
        pltpu.SemaphoreType.DMA,
        pltpu.SemaphoreType.DMA,
    ],
)
def _sc_gather(table_hbm, idx_hbm, out_hbm, idx_s, idx_sp, rows_v, sem, osem):
    wid = lax.axis_index("s") * _NC + lax.axis_index("c")
    base = wid * _BPW
    pltpu.sync_copy(idx_hbm.at[pl.ds(base, _BPW)], idx_sp.at[pl.ds(base, _BPW)])
    pltpu.sync_copy(idx_sp.at[pl.ds(base, _BPW)], idx_s)

    n_groups = _BPW // _K

    def issue(g):
        for j in range(_K):
            i = g * _K + j
            pltpu.async_copy(
                table_hbm.at[pl.ds(idx_s[i], 1)], rows_v.at[pl.ds(i, 1)], sem
            )

    def drain(g):
        pltpu.make_async_copy(
            table_hbm.at[pl.ds(0, _K)], rows_v.at[pl.ds(g * _K, _K)], sem
        ).wait()

    for g in range(_LAG):
        issue(g)

    def put(g):
        pltpu.async_copy(
            rows_v.at[pl.ds(g * _K, _K)],
            out_hbm.at[pl.ds(base + g * _K, _K)],
            osem,
        )

    def body(g, _):
        issue(g)
        drain(g - _LAG)
        put(g - _LAG)
        return ()

    lax.fori_loop(_LAG, n_groups, body, (), unroll=False)
    for g in range(n_groups - _LAG, n_groups):
        drain(g)
        put(g)
    for g in range(n_groups):
        pltpu.make_async_copy(
            rows_v.at[pl.ds(0, _K)],
            out_hbm.at[pl.ds(base, _K)],
            osem,
        ).wait()


def _mm_body(w_ref, e_ref, b_ref, o_ref):
    o_ref[...] = (
        lax.dot_general(
            w_ref[...], e_ref[...],
            (((0,), (1,)), ((), ())),
            preferred_element_type=jnp.float32,
        )
        + b_ref[...]
    )


def kernel(x, table, W, b):
    idx = x.reshape(B).astype(jnp.int32)
    e = _sc_gather(table, idx)
    blk = 8192
    out_t = pl.pallas_call(
        _mm_body,
        out_shape=jax.ShapeDtypeStruct((H, B), jnp.float32),
        grid=(B // blk,),
        in_specs=[
            pl.BlockSpec((H, H), lambda i: (0, 0)),
            pl.BlockSpec((blk, H), lambda i: (i, 0)),
            pl.BlockSpec((H, 1), lambda i: (0, 0)),
        ],
        out_specs=pl.BlockSpec((H, blk), lambda i: (0, i)),
    )(W, e, b.reshape(H, 1))
    return out_t.T
